# baseline (speedup 1.0000x reference)
import dataclasses
import functools

import jax
import jax.numpy as jnp
from jax import lax
from jax.experimental import pallas as pl
from jax.experimental.pallas import tpu as pltpu
from jax.experimental.pallas import tpu_sc as plsc

N = 10000
E = 320000
D = 128
H = 128
C = 40
CP = 48

NC = 2
NS = 16
NW = NC * NS
LANES = 16

NPAD = 10240

K1 = 80
NCHUNK1 = (E // NW) // K1
K2 = 128
E2 = NW * K2 * 80
NCHUNK2 = (E2 // NW) // K2
RPT = NPAD // NS

EPT_DEG = E // NS
SL = NPAD // NS

_mesh = plsc.VectorSubcoreMesh(core_axis_name="c", subcore_axis_name="s")

_sc_params = pltpu.CompilerParams()
for _f, _v in (("needs_layout_passes", False), ("use_tc_tiling_on_sc", False)):
    if _f in pltpu.CompilerParams.__dataclass_fields__:
        _sc_params = dataclasses.replace(_sc_params, **{_f: _v})


def _deg_dis_kernel(dst_hbm, dis_hbm, dst_v, deg_v, acc_v, tmp_v, deg_sh):
    cid = lax.axis_index("c")
    sid = lax.axis_index("s")

    @pl.when(cid == 0)
    def _():
        @pl.loop(0, NPAD, step=LANES)
        def _(i):
            deg_v[pl.ds(i, LANES)] = jnp.zeros((LANES,), jnp.float32)

        pltpu.sync_copy(dst_hbm.at[pl.ds(sid * EPT_DEG, EPT_DEG)], dst_v)

        @pl.loop(0, EPT_DEG, step=LANES)
        def _(i):
            idx = dst_v[pl.ds(i, LANES)]
            plsc.addupdate_scatter(deg_v, [idx], jnp.ones((LANES,), jnp.float32))

        pltpu.sync_copy(deg_v, deg_sh.at[sid])

    plsc.subcore_barrier()

    @pl.when(cid == 0)
    def _():
        @pl.loop(0, SL, step=LANES)
        def _(i):
            acc_v[pl.ds(i, LANES)] = jnp.zeros((LANES,), jnp.float32)

        @pl.loop(0, NS)
        def _(k):
            pltpu.sync_copy(deg_sh.at[k, pl.ds(sid * SL, SL)], tmp_v)

            @pl.loop(0, SL, step=LANES)
            def _(i):
                acc_v[pl.ds(i, LANES)] = acc_v[pl.ds(i, LANES)] + tmp_v[pl.ds(i, LANES)]

        @pl.loop(0, SL, step=LANES)
        def _(i):
            d = acc_v[pl.ds(i, LANES)] + 1.0
            xh = d * 0.5
            ii = plsc.bitcast(d, jnp.int32)
            ii = jnp.int32(0x5F3759DF) - lax.shift_right_logical(ii, jnp.int32(1))
            yv = plsc.bitcast(ii, jnp.float32)
            yv = yv * (1.5 - xh * yv * yv)
            yv = yv * (1.5 - xh * yv * yv)
            yv = yv * (1.5 - xh * yv * yv)
            acc_v[pl.ds(i, LANES)] = yv

        pltpu.sync_copy(acc_v, dis_hbm.at[pl.ds(sid * SL, SL)])


def _make_deg_dis():
    return pl.kernel(
        _deg_dis_kernel,
        out_type=jax.ShapeDtypeStruct((NPAD,), jnp.float32),
        mesh=_mesh,
        scratch_types=[
            pltpu.VMEM((EPT_DEG,), jnp.int32),
            pltpu.VMEM((NPAD,), jnp.float32),
            pltpu.VMEM((SL,), jnp.float32),
            pltpu.VMEM((SL,), jnp.float32),
            pltpu.VMEM_SHARED((NS, NPAD), jnp.float32),
        ],
        compiler_params=_sc_params,
    )


RING = 2


def _agg_kernel(nchunk, y_hbm, src_hbm, dst_hbm, zeros_hbm, out_hbm,
                src_v, dst_v, rows_v, acc_sh, gsem, ssem):
    cid = lax.axis_index("c")
    sid = lax.axis_index("s")
    wid = sid * NC + cid

    pltpu.sync_copy(src_hbm.at[wid], src_v)
    pltpu.sync_copy(dst_hbm.at[wid], dst_v)
    pltpu.sync_copy(zeros_hbm, acc_sh.at[pl.ds(sid * RPT, RPT)])
    plsc.subcore_barrier()

    main = (nchunk // RING) * RING

    for r in range(RING):
        pltpu.async_copy(y_hbm.at[src_v.at[r]], rows_v.at[r], gsem.at[r])

    @pl.loop(0, main, step=RING)
    def _(j0):
        for r in range(RING):
            j = j0 + r
            pltpu.make_async_copy(
                y_hbm.at[src_v.at[j]], rows_v.at[r], gsem.at[r]).wait()
            pltpu.async_copy(
                rows_v.at[r], acc_sh.at[dst_v.at[j]], ssem.at[r], add=True).wait()

            @pl.when(j + RING < main)
            def _():
                pltpu.async_copy(
                    y_hbm.at[src_v.at[j + RING]], rows_v.at[r], gsem.at[r])

    if main < nchunk:
        @pl.loop(main, nchunk)
        def _(j):
            pltpu.sync_copy(y_hbm.at[src_v.at[j]], rows_v.at[0])
            pltpu.sync_copy(rows_v.at[0], acc_sh.at[dst_v.at[j]], add=True)

    plsc.subcore_barrier()
    pltpu.sync_copy(acc_sh.at[pl.ds(sid * RPT, RPT)],
                    out_hbm.at[cid, pl.ds(sid * RPT, RPT)])


def _make_agg(width, k, nchunk):
    return pl.kernel(
        functools.partial(_agg_kernel, nchunk),
        out_type=jax.ShapeDtypeStruct((NC, NPAD, width), jnp.float32),
        mesh=_mesh,
        scratch_types=[
            pltpu.VMEM((nchunk, k), jnp.int32),
            pltpu.VMEM((nchunk, k), jnp.int32),
            pltpu.VMEM((RING, k, width), jnp.float32),
            pltpu.VMEM_SHARED((NPAD, width), jnp.float32),
            pltpu.SemaphoreType.DMA((RING,)),
            pltpu.SemaphoreType.DMA((RING,)),
        ],
        compiler_params=_sc_params,
    )


RB = 2048
GRID = NPAD // RB


def _tc1_body(dis_ref, x_ref, w_ref, y_ref):
    xw = jnp.dot(x_ref[...], w_ref[...], preferred_element_type=jnp.float32)
    y_ref[...] = xw * dis_ref[...]


def _tc2_body(dis_ref, p_ref, y_ref, b_ref, w_ref, z_ref):
    pre = dis_ref[...] * (p_ref[0] + p_ref[1] + y_ref[...]) + b_ref[...]
    h = jnp.maximum(pre, 0.0)
    z_ref[...] = jnp.dot(h, w_ref[...], preferred_element_type=jnp.float32) * dis_ref[...]


def _tc3_body(dis_ref, q_ref, z_ref, b_ref, o_ref):
    logits = dis_ref[...] * (q_ref[0] + q_ref[1] + z_ref[...]) + b_ref[...]
    m = jnp.max(logits, axis=1, keepdims=True)
    e = logits - m
    out = e - jnp.log(jnp.sum(jnp.exp(e), axis=1, keepdims=True))
    o_ref[...] = out[:, :C]


def _tc1(dis2d, x, W1):
    return pl.pallas_call(
        _tc1_body,
        grid=(GRID,),
        in_specs=[
            pl.BlockSpec((RB, 1), lambda i: (i, 0)),
            pl.BlockSpec((RB, D), lambda i: (i, 0)),
            pl.BlockSpec((D, H), lambda i: (0, 0)),
        ],
        out_specs=pl.BlockSpec((RB, H), lambda i: (i, 0)),
        out_shape=jax.ShapeDtypeStruct((NPAD, H), jnp.float32),
    )(dis2d, x, W1)


def _tc2(dis2d, p, y, b1r, W2p):
    return pl.pallas_call(
        _tc2_body,
        grid=(GRID,),
        in_specs=[
            pl.BlockSpec((RB, 1), lambda i: (i, 0)),
            pl.BlockSpec((NC, RB, H), lambda i: (0, i, 0)),
            pl.BlockSpec((RB, H), lambda i: (i, 0)),
            pl.BlockSpec((1, H), lambda i: (0, 0)),
            pl.BlockSpec((H, CP), lambda i: (0, 0)),
        ],
        out_specs=pl.BlockSpec((RB, CP), lambda i: (i, 0)),
        out_shape=jax.ShapeDtypeStruct((NPAD, CP), jnp.float32),
    )(dis2d, p, y, b1r, W2p)


RB3 = 2000


def _tc3(dis2d, q, z, b2r):
    return pl.pallas_call(
        _tc3_body,
        grid=(N // RB3,),
        in_specs=[
            pl.BlockSpec((RB3, 1), lambda i: (i, 0)),
            pl.BlockSpec((NC, RB3, CP), lambda i: (0, i, 0)),
            pl.BlockSpec((RB3, CP), lambda i: (i, 0)),
            pl.BlockSpec((1, CP), lambda i: (0, 0)),
        ],
        out_specs=pl.BlockSpec((RB3, C), lambda i: (i, 0)),
        out_shape=jax.ShapeDtypeStruct((N, C), jnp.float32),
    )(dis2d, q, z, b2r)


def kernel(x, edge_index, W1, b1, W2, b2):
    ei = edge_index.astype(jnp.int32)
    src1 = ei[0].reshape(NW, NCHUNK1, K1)
    dst1 = ei[1].reshape(NW, NCHUNK1, K1)
    padlen = E2 - E
    trash = jnp.full((padlen,), NPAD - 1, jnp.int32)
    src2 = jnp.concatenate([ei[0], trash]).reshape(NW, NCHUNK2, K2)
    dst2 = jnp.concatenate([ei[1], trash]).reshape(NW, NCHUNK2, K2)

    dis_pad = _make_deg_dis()(ei[1])
    dis2d = dis_pad.reshape(NPAD, 1)
    xp = jnp.pad(x, ((0, NPAD - N), (0, 0)))
    y = _tc1(dis2d, xp, W1)
    p = _make_agg(H, K1, NCHUNK1)(y, src1, dst1, jnp.zeros((RPT, H), jnp.float32))

    W2p = jnp.pad(W2, ((0, 0), (0, CP - C)))
    b1r = b1.reshape(1, H)
    z = _tc2(dis2d, p, y, b1r, W2p)

    q = _make_agg(CP, K2, NCHUNK2)(z, src2, dst2, jnp.zeros((RPT, CP), jnp.float32))
    b2r = jnp.concatenate([b2, jnp.full((CP - C,), -1e30, jnp.float32)]).reshape(1, CP)
    return _tc3(dis2d, q, z, b2r)


# speedup vs baseline: 1.3997x; 1.3997x over previous
import dataclasses
import functools

import jax
import jax.numpy as jnp
from jax import lax
from jax.experimental import pallas as pl
from jax.experimental.pallas import tpu as pltpu
from jax.experimental.pallas import tpu_sc as plsc

N = 10000
E = 320000
D = 128
H = 128
C = 40
CP = 48

NC = 2
NS = 16
NW = NC * NS
LANES = 16

NPAD = 10240

K1 = 80
NCHUNK1 = (E // NW) // K1
K2 = 128
E2 = NW * K2 * 80
NCHUNK2 = (E2 // NW) // K2
RPT = NPAD // NS

EPT_DEG = E // NS
SL = NPAD // NS

_mesh = plsc.VectorSubcoreMesh(core_axis_name="c", subcore_axis_name="s")

_sc_params = pltpu.CompilerParams()
for _f, _v in (("needs_layout_passes", False), ("use_tc_tiling_on_sc", False)):
    if _f in pltpu.CompilerParams.__dataclass_fields__:
        _sc_params = dataclasses.replace(_sc_params, **{_f: _v})


def _deg_dis_kernel(dst_hbm, dis_hbm, dst_v, deg_v, acc_v, tmp_v, deg_sh):
    cid = lax.axis_index("c")
    sid = lax.axis_index("s")

    @pl.when(cid == 0)
    def _():
        @pl.loop(0, NPAD, step=LANES)
        def _(i):
            deg_v[pl.ds(i, LANES)] = jnp.zeros((LANES,), jnp.float32)

        pltpu.sync_copy(dst_hbm.at[pl.ds(sid * EPT_DEG, EPT_DEG)], dst_v)

        @pl.loop(0, EPT_DEG, step=LANES)
        def _(i):
            idx = dst_v[pl.ds(i, LANES)]
            plsc.addupdate_scatter(deg_v, [idx], jnp.ones((LANES,), jnp.float32))

        pltpu.sync_copy(deg_v, deg_sh.at[sid])

    plsc.subcore_barrier()

    @pl.when(cid == 0)
    def _():
        @pl.loop(0, SL, step=LANES)
        def _(i):
            acc_v[pl.ds(i, LANES)] = jnp.zeros((LANES,), jnp.float32)

        @pl.loop(0, NS)
        def _(k):
            pltpu.sync_copy(deg_sh.at[k, pl.ds(sid * SL, SL)], tmp_v)

            @pl.loop(0, SL, step=LANES)
            def _(i):
                acc_v[pl.ds(i, LANES)] = acc_v[pl.ds(i, LANES)] + tmp_v[pl.ds(i, LANES)]

        @pl.loop(0, SL, step=LANES)
        def _(i):
            d = acc_v[pl.ds(i, LANES)] + 1.0
            xh = d * 0.5
            ii = plsc.bitcast(d, jnp.int32)
            ii = jnp.int32(0x5F3759DF) - lax.shift_right_logical(ii, jnp.int32(1))
            yv = plsc.bitcast(ii, jnp.float32)
            yv = yv * (1.5 - xh * yv * yv)
            yv = yv * (1.5 - xh * yv * yv)
            yv = yv * (1.5 - xh * yv * yv)
            acc_v[pl.ds(i, LANES)] = yv

        pltpu.sync_copy(acc_v, dis_hbm.at[pl.ds(sid * SL, SL)])


def _make_deg_dis():
    return pl.kernel(
        _deg_dis_kernel,
        out_type=jax.ShapeDtypeStruct((NPAD,), jnp.float32),
        mesh=_mesh,
        scratch_types=[
            pltpu.VMEM((EPT_DEG,), jnp.int32),
            pltpu.VMEM((NPAD,), jnp.float32),
            pltpu.VMEM((SL,), jnp.float32),
            pltpu.VMEM((SL,), jnp.float32),
            pltpu.VMEM_SHARED((NS, NPAD), jnp.float32),
        ],
        compiler_params=_sc_params,
    )


RING = 2


def _agg_kernel(nchunk, y_hbm, src_hbm, dst_hbm, zeros_hbm, out_hbm,
                src_v, dst_v, rows_v, acc_sh, gsem, ssem):
    cid = lax.axis_index("c")
    sid = lax.axis_index("s")
    wid = sid * NC + cid

    pltpu.sync_copy(src_hbm.at[wid], src_v)
    pltpu.sync_copy(dst_hbm.at[wid], dst_v)
    pltpu.sync_copy(zeros_hbm, acc_sh.at[pl.ds(sid * RPT, RPT)])
    plsc.subcore_barrier()

    main = (nchunk // RING) * RING

    for r in range(RING):
        pltpu.async_copy(y_hbm.at[src_v.at[r]], rows_v.at[r], gsem.at[r])

    @pl.loop(0, main, step=RING)
    def _(j0):
        for r in range(RING):
            j = j0 + r
            pltpu.make_async_copy(
                y_hbm.at[src_v.at[j]], rows_v.at[r], gsem.at[r]).wait()
            pltpu.async_copy(
                rows_v.at[r], acc_sh.at[dst_v.at[j]], ssem.at[r], add=True).wait()

            @pl.when(j + RING < main)
            def _():
                pltpu.async_copy(
                    y_hbm.at[src_v.at[j + RING]], rows_v.at[r], gsem.at[r])

    if main < nchunk:
        @pl.loop(main, nchunk)
        def _(j):
            pltpu.sync_copy(y_hbm.at[src_v.at[j]], rows_v.at[0])
            pltpu.sync_copy(rows_v.at[0], acc_sh.at[dst_v.at[j]], add=True)

    plsc.subcore_barrier()
    pltpu.sync_copy(acc_sh.at[pl.ds(sid * RPT, RPT)],
                    out_hbm.at[cid, pl.ds(sid * RPT, RPT)])


def _make_agg(width, k, nchunk):
    return pl.kernel(
        functools.partial(_agg_kernel, nchunk),
        out_type=jax.ShapeDtypeStruct((NC, NPAD, width), jnp.float32),
        mesh=_mesh,
        scratch_types=[
            pltpu.VMEM((nchunk, k), jnp.int32),
            pltpu.VMEM((nchunk, k), jnp.int32),
            pltpu.VMEM((RING, k, width), jnp.float32),
            pltpu.VMEM_SHARED((NPAD, width), jnp.float32),
            pltpu.SemaphoreType.DMA((RING,)),
            pltpu.SemaphoreType.DMA((RING,)),
        ],
        compiler_params=_sc_params,
    )


RB = 2048
GRID = NPAD // RB


def _tc1_body(dis_ref, x_ref, w_ref, y_ref):
    xw = jnp.dot(x_ref[...], w_ref[...], preferred_element_type=jnp.float32)
    y_ref[...] = xw * dis_ref[...]


def _tc2_body(dis_ref, p_ref, y_ref, b_ref, w_ref, z_ref):
    pre = dis_ref[...] * (p_ref[0] + p_ref[1] + y_ref[...]) + b_ref[...]
    h = jnp.maximum(pre, 0.0)
    z_ref[...] = jnp.dot(h, w_ref[...], preferred_element_type=jnp.float32) * dis_ref[...]


def _tc3_body(dis_ref, q_ref, z_ref, b_ref, o_ref):
    logits = dis_ref[...] * (q_ref[0] + q_ref[1] + z_ref[...]) + b_ref[...]
    m = jnp.max(logits, axis=1, keepdims=True)
    e = logits - m
    out = e - jnp.log(jnp.sum(jnp.exp(e), axis=1, keepdims=True))
    o_ref[...] = out[:, :C]


def _tc1(dis2d, x, W1):
    return pl.pallas_call(
        _tc1_body,
        grid=(GRID,),
        in_specs=[
            pl.BlockSpec((RB, 1), lambda i: (i, 0)),
            pl.BlockSpec((RB, D), lambda i: (i, 0)),
            pl.BlockSpec((D, H), lambda i: (0, 0)),
        ],
        out_specs=pl.BlockSpec((RB, H), lambda i: (i, 0)),
        out_shape=jax.ShapeDtypeStruct((NPAD, H), jnp.float32),
    )(dis2d, x, W1)


def _tc2(dis2d, p, y, b1r, W2p):
    return pl.pallas_call(
        _tc2_body,
        grid=(GRID,),
        in_specs=[
            pl.BlockSpec((RB, 1), lambda i: (i, 0)),
            pl.BlockSpec((NC, RB, H), lambda i: (0, i, 0)),
            pl.BlockSpec((RB, H), lambda i: (i, 0)),
            pl.BlockSpec((1, H), lambda i: (0, 0)),
            pl.BlockSpec((H, CP), lambda i: (0, 0)),
        ],
        out_specs=pl.BlockSpec((RB, CP), lambda i: (i, 0)),
        out_shape=jax.ShapeDtypeStruct((NPAD, CP), jnp.float32),
    )(dis2d, p, y, b1r, W2p)


RB3 = 2000


def _tc3(dis2d, q, z, b2r):
    return pl.pallas_call(
        _tc3_body,
        grid=(N // RB3,),
        in_specs=[
            pl.BlockSpec((RB3, 1), lambda i: (i, 0)),
            pl.BlockSpec((NC, RB3, CP), lambda i: (0, i, 0)),
            pl.BlockSpec((RB3, CP), lambda i: (i, 0)),
            pl.BlockSpec((1, CP), lambda i: (0, 0)),
        ],
        out_specs=pl.BlockSpec((RB3, C), lambda i: (i, 0)),
        out_shape=jax.ShapeDtypeStruct((N, C), jnp.float32),
    )(dis2d, q, z, b2r)


def kernel(x, edge_index, W1, b1, W2, b2):
    ei = edge_index.astype(jnp.int32)
    src1 = ei[0].reshape(NW, NCHUNK1, K1)
    dst1 = ei[1].reshape(NW, NCHUNK1, K1)
    padlen = E2 - E
    trash = N + (jnp.arange(padlen, dtype=jnp.int32) % (NPAD - N))
    src2 = jnp.concatenate([ei[0], trash]).reshape(NW, NCHUNK2, K2)
    dst2 = jnp.concatenate([ei[1], trash]).reshape(NW, NCHUNK2, K2)

    dis_pad = _make_deg_dis()(ei[1])
    dis2d = dis_pad.reshape(NPAD, 1)
    xp = jnp.pad(x, ((0, NPAD - N), (0, 0)))
    y = _tc1(dis2d, xp, W1)
    p = _make_agg(H, K1, NCHUNK1)(y, src1, dst1, jnp.zeros((RPT, H), jnp.float32))

    W2p = jnp.pad(W2, ((0, 0), (0, CP - C)))
    b1r = b1.reshape(1, H)
    z = _tc2(dis2d, p, y, b1r, W2p)

    q = _make_agg(CP, K2, NCHUNK2)(z, src2, dst2, jnp.zeros((RPT, CP), jnp.float32))
    b2r = jnp.concatenate([b2, jnp.full((CP - C,), -1e30, jnp.float32)]).reshape(1, CP)
    return _tc3(dis2d, q, z, b2r)


# speedup vs baseline: 1.4624x; 1.0447x over previous
import dataclasses
import functools

import jax
import jax.numpy as jnp
from jax import lax
from jax.experimental import pallas as pl
from jax.experimental.pallas import tpu as pltpu
from jax.experimental.pallas import tpu_sc as plsc

N = 10000
E = 320000
D = 128
H = 128
C = 40
CP = 48

NC = 2
NS = 16
NW = NC * NS
LANES = 16

NPAD = 10240

K1 = 80
NCHUNK1 = (E // NW) // K1
K2 = 128
E2 = NW * K2 * 80
NCHUNK2 = (E2 // NW) // K2
RPT = NPAD // NS

EPT_DEG = E // NS
SL = NPAD // NS

_mesh = plsc.VectorSubcoreMesh(core_axis_name="c", subcore_axis_name="s")

_sc_params = pltpu.CompilerParams()
for _f, _v in (("needs_layout_passes", False), ("use_tc_tiling_on_sc", False)):
    if _f in pltpu.CompilerParams.__dataclass_fields__:
        _sc_params = dataclasses.replace(_sc_params, **{_f: _v})


def _deg_dis_kernel(dst_hbm, dis_hbm, dst_v, deg_v, acc_v, tmp_v, deg_sh):
    cid = lax.axis_index("c")
    sid = lax.axis_index("s")

    @pl.when(cid == 0)
    def _():
        @pl.loop(0, NPAD, step=LANES)
        def _(i):
            deg_v[pl.ds(i, LANES)] = jnp.zeros((LANES,), jnp.float32)

        pltpu.sync_copy(dst_hbm.at[pl.ds(sid * EPT_DEG, EPT_DEG)], dst_v)

        @pl.loop(0, EPT_DEG, step=LANES)
        def _(i):
            idx = dst_v[pl.ds(i, LANES)]
            plsc.addupdate_scatter(deg_v, [idx], jnp.ones((LANES,), jnp.float32))

        pltpu.sync_copy(deg_v, deg_sh.at[sid])

    plsc.subcore_barrier()

    @pl.when(cid == 0)
    def _():
        @pl.loop(0, SL, step=LANES)
        def _(i):
            acc_v[pl.ds(i, LANES)] = jnp.zeros((LANES,), jnp.float32)

        @pl.loop(0, NS)
        def _(k):
            pltpu.sync_copy(deg_sh.at[k, pl.ds(sid * SL, SL)], tmp_v)

            @pl.loop(0, SL, step=LANES)
            def _(i):
                acc_v[pl.ds(i, LANES)] = acc_v[pl.ds(i, LANES)] + tmp_v[pl.ds(i, LANES)]

        @pl.loop(0, SL, step=LANES)
        def _(i):
            d = acc_v[pl.ds(i, LANES)] + 1.0
            xh = d * 0.5
            ii = plsc.bitcast(d, jnp.int32)
            ii = jnp.int32(0x5F3759DF) - lax.shift_right_logical(ii, jnp.int32(1))
            yv = plsc.bitcast(ii, jnp.float32)
            yv = yv * (1.5 - xh * yv * yv)
            yv = yv * (1.5 - xh * yv * yv)
            yv = yv * (1.5 - xh * yv * yv)
            acc_v[pl.ds(i, LANES)] = yv

        pltpu.sync_copy(acc_v, dis_hbm.at[pl.ds(sid * SL, SL)])


def _make_deg_dis():
    return pl.kernel(
        _deg_dis_kernel,
        out_type=jax.ShapeDtypeStruct((NPAD,), jnp.float32),
        mesh=_mesh,
        scratch_types=[
            pltpu.VMEM((EPT_DEG,), jnp.int32),
            pltpu.VMEM((NPAD,), jnp.float32),
            pltpu.VMEM((SL,), jnp.float32),
            pltpu.VMEM((SL,), jnp.float32),
            pltpu.VMEM_SHARED((NS, NPAD), jnp.float32),
        ],
        compiler_params=_sc_params,
    )


RING = 2


def _agg_kernel(nchunk, y_hbm, sd_hbm, zeros_hbm, out_hbm,
                sd_v, rows_v, acc_sh, gsem, ssem):
    cid = lax.axis_index("c")
    sid = lax.axis_index("s")
    wid = sid * NC + cid

    pltpu.sync_copy(sd_hbm.at[wid], sd_v)
    pltpu.sync_copy(zeros_hbm, acc_sh.at[pl.ds(sid * RPT, RPT)])
    plsc.subcore_barrier()

    main = (nchunk // RING) * RING

    for r in range(RING):
        pltpu.async_copy(y_hbm.at[sd_v.at[r, 0]], rows_v.at[r], gsem.at[r])

    @pl.loop(0, main, step=RING)
    def _(j0):
        for r in range(RING):
            j = j0 + r
            pltpu.make_async_copy(
                y_hbm.at[sd_v.at[j, 0]], rows_v.at[r], gsem.at[r]).wait()
            pltpu.async_copy(
                rows_v.at[r], acc_sh.at[sd_v.at[j, 1]], ssem.at[r],
                add=True).wait()

            @pl.when(j + RING < main)
            def _():
                pltpu.async_copy(
                    y_hbm.at[sd_v.at[j + RING, 0]], rows_v.at[r], gsem.at[r])

    if main < nchunk:
        @pl.loop(main, nchunk)
        def _(j):
            pltpu.sync_copy(y_hbm.at[sd_v.at[j, 0]], rows_v.at[0])
            pltpu.sync_copy(rows_v.at[0], acc_sh.at[sd_v.at[j, 1]], add=True)

    plsc.subcore_barrier()
    pltpu.sync_copy(acc_sh.at[pl.ds(sid * RPT, RPT)],
                    out_hbm.at[cid, pl.ds(sid * RPT, RPT)])


ISLOTS = 4


def _agg_stream_kernel(nchunk, y_hbm, sd_hbm, zeros_hbm, out_hbm,
                       sd_b, rows_v, acc_sh, gsem, ssem, isem):
    cid = lax.axis_index("c")
    sid = lax.axis_index("s")
    wid = sid * NC + cid

    pltpu.sync_copy(zeros_hbm, acc_sh.at[pl.ds(sid * RPT, RPT)])
    plsc.subcore_barrier()

    for s in range(ISLOTS):
        pltpu.async_copy(sd_hbm.at[wid, s], sd_b.at[s], isem.at[s])
    for r in range(RING):
        pltpu.make_async_copy(sd_hbm.at[wid, r], sd_b.at[r], isem.at[r]).wait()
        pltpu.async_copy(y_hbm.at[sd_b.at[r, 0]], rows_v.at[r], gsem.at[r])

    @pl.loop(0, nchunk, step=ISLOTS)
    def _(j0):
        for u in range(ISLOTS):
            j = j0 + u
            r = u % RING
            s2 = (u + RING) % ISLOTS
            pltpu.make_async_copy(
                y_hbm.at[sd_b.at[u, 0]], rows_v.at[r], gsem.at[r]).wait()
            pltpu.async_copy(
                rows_v.at[r], acc_sh.at[sd_b.at[u, 1]], ssem.at[r],
                add=True).wait()

            @pl.when(j + ISLOTS < nchunk)
            def _():
                pltpu.async_copy(sd_hbm.at[wid, j + ISLOTS], sd_b.at[u],
                                 isem.at[u])

            @pl.when(j + RING < nchunk)
            def _():
                pltpu.make_async_copy(
                    sd_hbm.at[wid, j + RING], sd_b.at[s2], isem.at[s2]).wait()
                pltpu.async_copy(
                    y_hbm.at[sd_b.at[s2, 0]], rows_v.at[r], gsem.at[r])

    plsc.subcore_barrier()
    pltpu.sync_copy(acc_sh.at[pl.ds(sid * RPT, RPT)],
                    out_hbm.at[cid, pl.ds(sid * RPT, RPT)])


def _make_agg_stream(width, k, nchunk):
    return pl.kernel(
        functools.partial(_agg_stream_kernel, nchunk),
        out_type=jax.ShapeDtypeStruct((NC, NPAD, width), jnp.float32),
        mesh=_mesh,
        scratch_types=[
            pltpu.VMEM((ISLOTS, 2, k), jnp.int32),
            pltpu.VMEM((RING, k, width), jnp.float32),
            pltpu.VMEM_SHARED((NPAD, width), jnp.float32),
            pltpu.SemaphoreType.DMA((RING,)),
            pltpu.SemaphoreType.DMA((RING,)),
            pltpu.SemaphoreType.DMA((ISLOTS,)),
        ],
        compiler_params=_sc_params,
    )


def _make_agg(width, k, nchunk):
    return pl.kernel(
        functools.partial(_agg_kernel, nchunk),
        out_type=jax.ShapeDtypeStruct((NC, NPAD, width), jnp.float32),
        mesh=_mesh,
        scratch_types=[
            pltpu.VMEM((nchunk, 2, k), jnp.int32),
            pltpu.VMEM((RING, k, width), jnp.float32),
            pltpu.VMEM_SHARED((NPAD, width), jnp.float32),
            pltpu.SemaphoreType.DMA((RING,)),
            pltpu.SemaphoreType.DMA((RING,)),
        ],
        compiler_params=_sc_params,
    )


RB = 2048
GRID = NPAD // RB


def _tc1_body(dis_ref, x_ref, w_ref, y_ref):
    xw = jnp.dot(x_ref[...], w_ref[...], preferred_element_type=jnp.float32)
    y_ref[...] = xw * dis_ref[...]


def _tc2_body(dis_ref, p_ref, y_ref, b_ref, w_ref, z_ref):
    pre = dis_ref[...] * (p_ref[0] + p_ref[1] + y_ref[...]) + b_ref[...]
    h = jnp.maximum(pre, 0.0)
    z_ref[...] = jnp.dot(h, w_ref[...], preferred_element_type=jnp.float32) * dis_ref[...]


def _tc3_body(dis_ref, q_ref, z_ref, b_ref, o_ref):
    logits = dis_ref[...] * (q_ref[0] + q_ref[1] + z_ref[...]) + b_ref[...]
    m = jnp.max(logits, axis=1, keepdims=True)
    e = logits - m
    out = e - jnp.log(jnp.sum(jnp.exp(e), axis=1, keepdims=True))
    o_ref[...] = out[:, :C]


def _tc1(dis2d, x, W1):
    return pl.pallas_call(
        _tc1_body,
        grid=(GRID,),
        in_specs=[
            pl.BlockSpec((RB, 1), lambda i: (i, 0)),
            pl.BlockSpec((RB, D), lambda i: (i, 0)),
            pl.BlockSpec((D, H), lambda i: (0, 0)),
        ],
        out_specs=pl.BlockSpec((RB, H), lambda i: (i, 0)),
        out_shape=jax.ShapeDtypeStruct((NPAD, H), jnp.float32),
    )(dis2d, x, W1)


def _tc2(dis2d, p, y, b1r, W2p):
    return pl.pallas_call(
        _tc2_body,
        grid=(GRID,),
        in_specs=[
            pl.BlockSpec((RB, 1), lambda i: (i, 0)),
            pl.BlockSpec((NC, RB, H), lambda i: (0, i, 0)),
            pl.BlockSpec((RB, H), lambda i: (i, 0)),
            pl.BlockSpec((1, H), lambda i: (0, 0)),
            pl.BlockSpec((H, CP), lambda i: (0, 0)),
        ],
        out_specs=pl.BlockSpec((RB, CP), lambda i: (i, 0)),
        out_shape=jax.ShapeDtypeStruct((NPAD, CP), jnp.float32),
    )(dis2d, p, y, b1r, W2p)


RB3 = 2000


def _tc3(dis2d, q, z, b2r):
    return pl.pallas_call(
        _tc3_body,
        grid=(N // RB3,),
        in_specs=[
            pl.BlockSpec((RB3, 1), lambda i: (i, 0)),
            pl.BlockSpec((NC, RB3, CP), lambda i: (0, i, 0)),
            pl.BlockSpec((RB3, CP), lambda i: (i, 0)),
            pl.BlockSpec((1, CP), lambda i: (0, 0)),
        ],
        out_specs=pl.BlockSpec((RB3, C), lambda i: (i, 0)),
        out_shape=jax.ShapeDtypeStruct((N, C), jnp.float32),
    )(dis2d, q, z, b2r)


def kernel(x, edge_index, W1, b1, W2, b2):
    ei = edge_index.astype(jnp.int32)
    padlen = E2 - E
    trash = N + (jnp.arange(padlen, dtype=jnp.int32) % (NPAD - N))
    srcp = jnp.concatenate([ei[0], trash]).reshape(NW, NCHUNK2, K2)
    dstp = jnp.concatenate([ei[1], trash]).reshape(NW, NCHUNK2, K2)
    sd3 = jnp.stack([srcp, dstp], axis=2)

    dis_pad = _make_deg_dis()(ei[1])
    dis2d = dis_pad.reshape(NPAD, 1)
    xp = jnp.pad(x, ((0, NPAD - N), (0, 0)))
    y = _tc1(dis2d, xp, W1)
    p = _make_agg_stream(H, K2, NCHUNK2)(y, sd3, jnp.zeros((RPT, H), jnp.float32))

    W2p = jnp.pad(W2, ((0, 0), (0, CP - C)))
    b1r = b1.reshape(1, H)
    z = _tc2(dis2d, p, y, b1r, W2p)

    q = _make_agg(CP, K2, NCHUNK2)(z, sd3, jnp.zeros((RPT, CP), jnp.float32))
    b2r = jnp.concatenate([b2, jnp.full((CP - C,), -1e30, jnp.float32)]).reshape(1, CP)
    return _tc3(dis2d, q, z, b2r)


# speedup vs baseline: 1.4678x; 1.0037x over previous
import dataclasses
import functools

import jax
import jax.numpy as jnp
from jax import lax
from jax.experimental import pallas as pl
from jax.experimental.pallas import tpu as pltpu
from jax.experimental.pallas import tpu_sc as plsc

N = 10000
E = 320000
D = 128
H = 128
C = 40
CP = 48

NC = 2
NS = 16
NW = NC * NS
LANES = 16

NPAD = 10240

K2 = 128
E2 = NW * K2 * 80
NCHUNK2 = (E2 // NW) // K2
RPT = NPAD // NS

EPT_DEG = E // NS
SL = NPAD // NS

_mesh = plsc.VectorSubcoreMesh(core_axis_name="c", subcore_axis_name="s")

_sc_params = pltpu.CompilerParams()
for _f, _v in (("needs_layout_passes", False), ("use_tc_tiling_on_sc", False)):
    if _f in pltpu.CompilerParams.__dataclass_fields__:
        _sc_params = dataclasses.replace(_sc_params, **{_f: _v})


def _deg_dis_kernel(dst_hbm, dis_hbm, dst_v, deg_v, acc_v, tmp_v, deg_sh):
    cid = lax.axis_index("c")
    sid = lax.axis_index("s")

    @pl.when(cid == 0)
    def _():
        @pl.loop(0, NPAD, step=LANES)
        def _(i):
            deg_v[pl.ds(i, LANES)] = jnp.zeros((LANES,), jnp.float32)

        pltpu.sync_copy(dst_hbm.at[pl.ds(sid * EPT_DEG, EPT_DEG)], dst_v)

        @pl.loop(0, EPT_DEG, step=LANES)
        def _(i):
            idx = dst_v[pl.ds(i, LANES)]
            plsc.addupdate_scatter(deg_v, [idx], jnp.ones((LANES,), jnp.float32))

        pltpu.sync_copy(deg_v, deg_sh.at[sid])

    plsc.subcore_barrier()

    @pl.when(cid == 0)
    def _():
        @pl.loop(0, SL, step=LANES)
        def _(i):
            acc_v[pl.ds(i, LANES)] = jnp.zeros((LANES,), jnp.float32)

        @pl.loop(0, NS)
        def _(k):
            pltpu.sync_copy(deg_sh.at[k, pl.ds(sid * SL, SL)], tmp_v)

            @pl.loop(0, SL, step=LANES)
            def _(i):
                acc_v[pl.ds(i, LANES)] = acc_v[pl.ds(i, LANES)] + tmp_v[pl.ds(i, LANES)]

        @pl.loop(0, SL, step=LANES)
        def _(i):
            d = acc_v[pl.ds(i, LANES)] + 1.0
            xh = d * 0.5
            ii = plsc.bitcast(d, jnp.int32)
            ii = jnp.int32(0x5F3759DF) - lax.shift_right_logical(ii, jnp.int32(1))
            yv = plsc.bitcast(ii, jnp.float32)
            yv = yv * (1.5 - xh * yv * yv)
            yv = yv * (1.5 - xh * yv * yv)
            yv = yv * (1.5 - xh * yv * yv)
            acc_v[pl.ds(i, LANES)] = yv

        pltpu.sync_copy(acc_v, dis_hbm.at[pl.ds(sid * SL, SL)])


def _make_deg_dis():
    return pl.kernel(
        _deg_dis_kernel,
        out_type=jax.ShapeDtypeStruct((NPAD,), jnp.float32),
        mesh=_mesh,
        scratch_types=[
            pltpu.VMEM((EPT_DEG,), jnp.int32),
            pltpu.VMEM((NPAD,), jnp.float32),
            pltpu.VMEM((SL,), jnp.float32),
            pltpu.VMEM((SL,), jnp.float32),
            pltpu.VMEM_SHARED((NS, NPAD), jnp.float32),
        ],
        compiler_params=_sc_params,
    )


RING = 2


def _agg_kernel(nchunk, y_hbm, sd_hbm, zeros_hbm, out_hbm,
                sd_v, rows_v, acc_sh, gsem, ssem):
    cid = lax.axis_index("c")
    sid = lax.axis_index("s")
    wid = sid * NC + cid

    pltpu.sync_copy(sd_hbm.at[wid], sd_v)
    pltpu.sync_copy(zeros_hbm, acc_sh.at[pl.ds(sid * RPT, RPT)])
    plsc.subcore_barrier()

    main = (nchunk // RING) * RING

    for r in range(RING):
        pltpu.async_copy(y_hbm.at[sd_v.at[r, 0]], rows_v.at[r], gsem.at[r])

    @pl.loop(0, main, step=RING)
    def _(j0):
        for r in range(RING):
            j = j0 + r
            pltpu.make_async_copy(
                y_hbm.at[sd_v.at[j, 0]], rows_v.at[r], gsem.at[r]).wait()
            pltpu.async_copy(
                rows_v.at[r], acc_sh.at[sd_v.at[j, 1]], ssem.at[r],
                add=True).wait()

            @pl.when(j + RING < main)
            def _():
                pltpu.async_copy(
                    y_hbm.at[sd_v.at[j + RING, 0]], rows_v.at[r], gsem.at[r])

    if main < nchunk:
        @pl.loop(main, nchunk)
        def _(j):
            pltpu.sync_copy(y_hbm.at[sd_v.at[j, 0]], rows_v.at[0])
            pltpu.sync_copy(rows_v.at[0], acc_sh.at[sd_v.at[j, 1]], add=True)

    plsc.subcore_barrier()
    pltpu.sync_copy(acc_sh.at[pl.ds(sid * RPT, RPT)],
                    out_hbm.at[cid, pl.ds(sid * RPT, RPT)])


ISLOTS = 4


def _agg_stream_kernel(nchunk, y_hbm, sd_hbm, zeros_hbm, out_hbm,
                       sd_b, rows_v, acc_sh, gsem, ssem, isem):
    cid = lax.axis_index("c")
    sid = lax.axis_index("s")
    wid = sid * NC + cid

    pltpu.sync_copy(zeros_hbm, acc_sh.at[pl.ds(sid * RPT, RPT)])
    plsc.subcore_barrier()

    for s in range(ISLOTS):
        pltpu.async_copy(sd_hbm.at[wid, s], sd_b.at[s], isem.at[s])
    for r in range(RING):
        pltpu.make_async_copy(sd_hbm.at[wid, r], sd_b.at[r], isem.at[r]).wait()
        pltpu.async_copy(y_hbm.at[sd_b.at[r, 0]], rows_v.at[r], gsem.at[r])

    @pl.loop(0, nchunk, step=ISLOTS)
    def _(j0):
        for u in range(ISLOTS):
            j = j0 + u
            r = u % RING
            s2 = (u + RING) % ISLOTS
            pltpu.make_async_copy(
                y_hbm.at[sd_b.at[u, 0]], rows_v.at[r], gsem.at[r]).wait()
            pltpu.async_copy(
                rows_v.at[r], acc_sh.at[sd_b.at[u, 1]], ssem.at[r],
                add=True).wait()

            @pl.when(j + ISLOTS < nchunk)
            def _():
                pltpu.async_copy(sd_hbm.at[wid, j + ISLOTS], sd_b.at[u],
                                 isem.at[u])

            @pl.when(j + RING < nchunk)
            def _():
                pltpu.make_async_copy(
                    sd_hbm.at[wid, j + RING], sd_b.at[s2], isem.at[s2]).wait()
                pltpu.async_copy(
                    y_hbm.at[sd_b.at[s2, 0]], rows_v.at[r], gsem.at[r])

    plsc.subcore_barrier()
    pltpu.sync_copy(acc_sh.at[pl.ds(sid * RPT, RPT)],
                    out_hbm.at[cid, pl.ds(sid * RPT, RPT)])


def _make_agg_stream(width, k, nchunk):
    return pl.kernel(
        functools.partial(_agg_stream_kernel, nchunk),
        out_type=jax.ShapeDtypeStruct((NC, NPAD, width), jnp.float32),
        mesh=_mesh,
        scratch_types=[
            pltpu.VMEM((ISLOTS, 2, k), jnp.int32),
            pltpu.VMEM((RING, k, width), jnp.float32),
            pltpu.VMEM_SHARED((NPAD, width), jnp.float32),
            pltpu.SemaphoreType.DMA((RING,)),
            pltpu.SemaphoreType.DMA((RING,)),
            pltpu.SemaphoreType.DMA((ISLOTS,)),
        ],
        compiler_params=_sc_params,
    )


def _make_agg(width, k, nchunk):
    return pl.kernel(
        functools.partial(_agg_kernel, nchunk),
        out_type=jax.ShapeDtypeStruct((NC, NPAD, width), jnp.float32),
        mesh=_mesh,
        scratch_types=[
            pltpu.VMEM((nchunk, 2, k), jnp.int32),
            pltpu.VMEM((RING, k, width), jnp.float32),
            pltpu.VMEM_SHARED((NPAD, width), jnp.float32),
            pltpu.SemaphoreType.DMA((RING,)),
            pltpu.SemaphoreType.DMA((RING,)),
        ],
        compiler_params=_sc_params,
    )


RB = 2048
GRID = NPAD // RB


def _tc1_body(dis_ref, x_ref, w_ref, y_ref):
    xw = jnp.dot(x_ref[...], w_ref[...], preferred_element_type=jnp.float32)
    y_ref[...] = xw * dis_ref[...]


def _tc2_body(dis_ref, p_ref, y_ref, b_ref, w_ref, z_ref):
    pre = dis_ref[...] * (p_ref[0] + p_ref[1] + y_ref[...]) + b_ref[...]
    h = jnp.maximum(pre, 0.0)
    z_ref[...] = jnp.dot(h, w_ref[...], preferred_element_type=jnp.float32) * dis_ref[...]


def _tc3_body(dis_ref, q_ref, z_ref, b_ref, o_ref):
    logits = dis_ref[...] * (q_ref[0] + q_ref[1] + z_ref[...]) + b_ref[...]
    m = jnp.max(logits, axis=1, keepdims=True)
    e = logits - m
    out = e - jnp.log(jnp.sum(jnp.exp(e), axis=1, keepdims=True))
    o_ref[...] = out[:, :C]


def _tc1(dis2d, x, W1):
    return pl.pallas_call(
        _tc1_body,
        grid=(GRID,),
        in_specs=[
            pl.BlockSpec((RB, 1), lambda i: (i, 0)),
            pl.BlockSpec((RB, D), lambda i: (i, 0)),
            pl.BlockSpec((D, H), lambda i: (0, 0)),
        ],
        out_specs=pl.BlockSpec((RB, H), lambda i: (i, 0)),
        out_shape=jax.ShapeDtypeStruct((NPAD, H), jnp.float32),
    )(dis2d, x, W1)


def _tc2(dis2d, p, y, b1r, W2p):
    return pl.pallas_call(
        _tc2_body,
        grid=(GRID,),
        in_specs=[
            pl.BlockSpec((RB, 1), lambda i: (i, 0)),
            pl.BlockSpec((NC, RB, H), lambda i: (0, i, 0)),
            pl.BlockSpec((RB, H), lambda i: (i, 0)),
            pl.BlockSpec((1, H), lambda i: (0, 0)),
            pl.BlockSpec((H, CP), lambda i: (0, 0)),
        ],
        out_specs=pl.BlockSpec((RB, CP), lambda i: (i, 0)),
        out_shape=jax.ShapeDtypeStruct((NPAD, CP), jnp.float32),
    )(dis2d, p, y, b1r, W2p)


RB3 = 2000


def _tc3(dis2d, q, z, b2r):
    return pl.pallas_call(
        _tc3_body,
        grid=(N // RB3,),
        in_specs=[
            pl.BlockSpec((RB3, 1), lambda i: (i, 0)),
            pl.BlockSpec((NC, RB3, CP), lambda i: (0, i, 0)),
            pl.BlockSpec((RB3, CP), lambda i: (i, 0)),
            pl.BlockSpec((1, CP), lambda i: (0, 0)),
        ],
        out_specs=pl.BlockSpec((RB3, C), lambda i: (i, 0)),
        out_shape=jax.ShapeDtypeStruct((N, C), jnp.float32),
    )(dis2d, q, z, b2r)


def kernel(x, edge_index, W1, b1, W2, b2):
    ei = edge_index.astype(jnp.int32)
    padlen = E2 - E
    trash = N + (jnp.arange(padlen, dtype=jnp.int32) % (NPAD - N))
    srcp = jnp.concatenate([ei[0], trash]).reshape(NW, NCHUNK2, K2)
    dstp = jnp.concatenate([ei[1], trash]).reshape(NW, NCHUNK2, K2)
    sd3 = jnp.stack([srcp, dstp], axis=2)

    dis_pad = _make_deg_dis()(ei[1])
    dis2d = dis_pad.reshape(NPAD, 1)
    xp = jnp.pad(x, ((0, NPAD - N), (0, 0)))
    y = _tc1(dis2d, xp, W1)
    p = _make_agg_stream(H, K2, NCHUNK2)(y, sd3, jnp.zeros((RPT, H), jnp.float32))

    W2p = jnp.pad(W2, ((0, 0), (0, CP - C)))
    b1r = b1.reshape(1, H)
    z = _tc2(dis2d, p, y, b1r, W2p)

    q = _make_agg(CP, K2, NCHUNK2)(z, sd3, jnp.zeros((RPT, CP), jnp.float32))
    b2r = jnp.concatenate([b2, jnp.full((CP - C,), -1e30, jnp.float32)]).reshape(1, CP)
    return _tc3(dis2d, q, z, b2r)
